# baseline (speedup 1.0000x reference)
import functools

import jax
import jax.numpy as jnp
from jax import lax
from jax.experimental import pallas as pl
from jax.experimental.pallas import tpu as pltpu
from jax.experimental.pallas import tpu_sc as plsc

_NUM_CORES = 2
_NUM_SUBCORES = 16
_NUM_WORKERS = _NUM_CORES * _NUM_SUBCORES

_CH = 1024


@functools.lru_cache(maxsize=None)
def _make_lookup(num_emb, dim, n_i, n_j):
    batch = n_i * n_j
    b_per_w = batch // _NUM_WORKERS
    n_chunks = b_per_w // _CH
    units_per_chunk = _CH // 128
    n_db = dim // 8
    n_ib = n_i // 128
    out_rows = n_j * n_db * n_ib
    mesh = plsc.VectorSubcoreMesh(core_axis_name="c", subcore_axis_name="s")

    @functools.partial(
        pl.kernel,
        out_type=jax.ShapeDtypeStruct((out_rows, 1024), jnp.float32),
        mesh=mesh,
        scratch_types=[
            pltpu.VMEM((b_per_w,), jnp.int32),
            pltpu.VMEM((_CH, dim), jnp.float32),
            pltpu.VMEM((n_db, units_per_chunk, 1024), jnp.float32),
            pltpu.SemaphoreType.DMA,
        ],
        compiler_params=pltpu.CompilerParams(
            use_tc_tiling_on_sc=False, needs_layout_passes=False
        ),
    )
    def lookup(ids_hbm, table_hbm, out_hbm, idx_v, rows_v, tbuf, sem):
        wid = lax.axis_index("s") * _NUM_CORES + lax.axis_index("c")
        u_base = wid * (b_per_w // 128)
        pltpu.sync_copy(ids_hbm.at[pl.ds(wid * b_per_w, b_per_w)], idx_v)
        lane = lax.iota(jnp.int32, 16)

        def chunk_body(c, carry):
            u0 = u_base + c * units_per_chunk
            j = u0 // n_ib
            ib0 = u0 % n_ib
            pltpu.async_copy(
                table_hbm.at[idx_v.at[pl.ds(c * _CH, _CH)]], rows_v, sem
            ).wait()
            for db in range(n_db):
                db_v = jnp.full((16,), db, jnp.int32)

                def tbody(g, carry2):
                    ib_l = g >> 3
                    di = g & 7
                    col = jnp.full((16,), db * 8 + di, jnp.int32)
                    ibl_v = jnp.full((16,), ib_l, jnp.int32)
                    for ii in range(8):
                        tok = jnp.full(
                            (16,), ib_l * 128 + ii * 16, jnp.int32) + lane
                        vals = plsc.load_gather(rows_v, [tok, col])
                        pos = jnp.full(
                            (16,), di * 128 + ii * 16, jnp.int32) + lane
                        plsc.store_scatter(tbuf, [db_v, ibl_v, pos], vals)
                    return carry2

                lax.fori_loop(0, units_per_chunk * 8, tbody, 0)
                r0 = j * (n_db * n_ib) + db * n_ib + ib0
                pltpu.sync_copy(
                    tbuf.at[db], out_hbm.at[pl.ds(r0, units_per_chunk), :]
                )
            return carry

        lax.fori_loop(0, n_chunks, chunk_body, 0)

    return lookup


def kernel(token_ids, weight):
    n_i, n_j = token_ids.shape
    num_emb, dim = weight.shape
    ids_t = token_ids.T.reshape(n_i * n_j).astype(jnp.int32)
    out2d = _make_lookup(num_emb, dim, n_i, n_j)(ids_t, weight)
    out5d = out2d.reshape(n_j, dim // 8, n_i // 128, 8, 128)
    return out5d.transpose(2, 4, 0, 1, 3).reshape(n_i, n_j, dim)


# speedup vs baseline: 1.1522x; 1.1522x over previous
import functools

import jax
import jax.numpy as jnp
from jax import lax
from jax.experimental import pallas as pl
from jax.experimental.pallas import tpu as pltpu
from jax.experimental.pallas import tpu_sc as plsc

_NUM_CORES = 2
_NUM_SUBCORES = 16
_NUM_WORKERS = _NUM_CORES * _NUM_SUBCORES

_CH = 1024


@functools.lru_cache(maxsize=None)
def _make_lookup(num_emb, dim, n_i, n_j):
    batch = n_i * n_j
    b_per_w = batch // _NUM_WORKERS
    n_chunks = b_per_w // _CH
    units_per_chunk = _CH // 128
    n_db = dim // 8
    n_ib = n_i // 128
    out_rows = n_j * n_db * n_ib
    mesh = plsc.VectorSubcoreMesh(core_axis_name="c", subcore_axis_name="s")

    @functools.partial(
        pl.kernel,
        out_type=jax.ShapeDtypeStruct((out_rows, 1024), jnp.float32),
        mesh=mesh,
        scratch_types=[
            pltpu.VMEM((b_per_w,), jnp.int32),
            pltpu.VMEM((_CH, dim), jnp.float32),
            pltpu.VMEM((n_db * units_per_chunk, 1024), jnp.float32),
            pltpu.SemaphoreType.DMA,
        ],
        compiler_params=pltpu.CompilerParams(
            use_tc_tiling_on_sc=False, needs_layout_passes=False
        ),
    )
    def lookup(ids_hbm, table_hbm, out_hbm, idx_v, rows_v, tbuf, sem):
        wid = lax.axis_index("s") * _NUM_CORES + lax.axis_index("c")
        u_base = wid * (b_per_w // 128)
        pltpu.sync_copy(ids_hbm.at[pl.ds(wid * b_per_w, b_per_w)], idx_v)
        n_half = dim // 16
        lane = lax.iota(jnp.int32, 16)
        col_pat = (lane & 7) * 128
        row_base = (lane >> 3) * units_per_chunk

        def chunk_body(c, carry):
            u0 = u_base + c * units_per_chunk
            j = u0 // n_ib
            ib0 = u0 % n_ib
            pltpu.async_copy(
                table_hbm.at[idx_v.at[pl.ds(c * _CH, _CH)]], rows_v, sem
            ).wait()
            for ib_l in range(units_per_chunk):
                row_pats = [
                    row_base + (2 * h * units_per_chunk + ib_l)
                    for h in range(n_half)
                ]

                @plsc.parallel_loop(0, 128, unroll=4)
                def _(ii):
                    t = ib_l * 128 + ii
                    for h in range(n_half):
                        vals = rows_v[t, pl.ds(h * 16, 16)]
                        plsc.store_scatter(
                            tbuf, [row_pats[h], col_pat + ii], vals
                        )

            for db in range(n_db):
                r0 = j * (n_db * n_ib) + db * n_ib + ib0
                pltpu.sync_copy(
                    tbuf.at[pl.ds(db * units_per_chunk, units_per_chunk)],
                    out_hbm.at[pl.ds(r0, units_per_chunk), :],
                )
            return carry

        lax.fori_loop(0, n_chunks, chunk_body, 0)

    return lookup


def kernel(token_ids, weight):
    n_i, n_j = token_ids.shape
    num_emb, dim = weight.shape
    ids_t = token_ids.T.reshape(n_i * n_j).astype(jnp.int32)
    out2d = _make_lookup(num_emb, dim, n_i, n_j)(ids_t, weight)
    out5d = out2d.reshape(n_j, dim // 8, n_i // 128, 8, 128)
    return out5d.transpose(2, 4, 0, 1, 3).reshape(n_i, n_j, dim)
